# baseline (speedup 1.0000x reference)
import functools

import jax
import jax.numpy as jnp
from jax import lax
from jax.experimental import pallas as pl
from jax.experimental.pallas import tpu as pltpu
from jax.experimental.pallas import tpu_sc as plsc

INTERPRET = False


def _lrelu(v):
    return jnp.where(v > 0, v, v * 0.2)


def _dot(a, b, prec=None):
    return jnp.dot(a, b, preferred_element_type=jnp.float32, precision=prec)


_SC_CH = 128


def _sc_gather(table, idx_flat):
    e = idx_flat.shape[0]
    c = table.shape[1]
    nw = 32
    per_w = e // nw
    n_chunks = per_w // _SC_CH
    mesh = plsc.VectorSubcoreMesh(core_axis_name="c", subcore_axis_name="s")

    @functools.partial(
        pl.kernel,
        mesh=mesh,
        out_type=jax.ShapeDtypeStruct((e, c), jnp.float32),
        scratch_types=[
            pltpu.VMEM((_SC_CH,), jnp.int32),
            pltpu.VMEM((_SC_CH, c), jnp.float32),
            pltpu.SemaphoreType.DMA,
        ],
        compiler_params=pltpu.CompilerParams(use_tc_tiling_on_sc=False),
    )
    def gk(table_hbm, idx_hbm, out_hbm, idx_v, rows_v, sem):
        wid = lax.axis_index("s") * 2 + lax.axis_index("c")
        base0 = wid * per_w

        def body(gi, carry):
            base = base0 + gi * _SC_CH
            pltpu.sync_copy(idx_hbm.at[pl.ds(base, _SC_CH)], idx_v)
            pltpu.async_copy(table_hbm.at[idx_v], rows_v, sem).wait()
            pltpu.sync_copy(rows_v, out_hbm.at[pl.ds(base, _SC_CH)])
            return carry

        lax.fori_loop(0, n_chunks, body, 0)

    return gk(table, idx_flat)


def _edge_body(k, x_ref, g_ref, wn_ref, wc_ref, b_ref, o_ref):
    x = x_ref[...]
    zc = _dot(x, wc_ref[...]) + b_ref[...]
    m = None
    for kk in range(k):
        h = _dot(g_ref[kk] - x, wn_ref[...]) + zc
        m = h if m is None else jnp.maximum(m, h)
    o_ref[...] = _lrelu(m)


def _edge_conv(x, g, wn, wc, b, k, br=512):
    r, c = x.shape
    d = wn.shape[1]
    grid = r // br
    body = functools.partial(_edge_body, k)
    return pl.pallas_call(
        body,
        grid=(grid,),
        in_specs=[
            pl.BlockSpec((br, c), lambda i: (i, 0)),
            pl.BlockSpec((k, br, c), lambda i: (0, i, 0)),
            pl.BlockSpec((c, d), lambda i: (0, 0)),
            pl.BlockSpec((c, d), lambda i: (0, 0)),
            pl.BlockSpec((1, d), lambda i: (0, 0)),
        ],
        out_specs=pl.BlockSpec((br, d), lambda i: (i, 0)),
        out_shape=jax.ShapeDtypeStruct((r, d), jnp.float32),
        interpret=INTERPRET,
    )(x, g.reshape(k, r, c), wn, wc, b.reshape(1, d))


def _a_body(wm_ref, f_ref, a_ref):
    wm = wm_ref[0]
    a = _dot(wm, f_ref[0])
    s = jnp.sum(wm, axis=-1, keepdims=True) + 1e-05
    a_ref[0] = a / s


def _a_matmul(wm, f):
    p, j, n = wm.shape
    d = f.shape[-1]
    return pl.pallas_call(
        _a_body,
        grid=(p,),
        in_specs=[
            pl.BlockSpec((1, j, n), lambda i: (i, 0, 0)),
            pl.BlockSpec((1, n, d), lambda i: (i, 0, 0)),
        ],
        out_specs=pl.BlockSpec((1, j, d), lambda i: (i, 0, 0)),
        out_shape=jax.ShapeDtypeStruct((p, j, d), jnp.float32),
        interpret=INTERPRET,
    )(wm, f)


def _mega_body(nres, cat_ref, o192_ref, o96_ref, *refs):
    refs = list(refs)
    out_ref = refs.pop()
    cur = [0]

    def take():
        v = refs[cur[0]]
        cur[0] += 1
        return v

    def conv(x, act=True):
        w, b = take()[...], take()[...]
        v = _dot(x, w) + b
        return _lrelu(v) if act else v

    def triple(x, o_ref):
        outs = []
        for _ in range(3):
            c = x.shape[-1]
            w, b = take()[...], take()[...]
            wn, wc = w[:c], w[c:]
            zc = _dot(x, wc) + b
            m = None
            for kk in range(4):
                nb = _dot(o_ref[kk], x, prec=lax.Precision.HIGHEST)
                h = _dot(nb - x, wn) + zc
                m = h if m is None else jnp.maximum(m, h)
            x = _lrelu(m)
            outs.append(x)
        return jnp.concatenate(outs, axis=-1)

    f192 = triple(cat_ref[...], o192_ref)
    x = jnp.concatenate([f192[:96], f192[96:]], axis=-1)
    x = conv(x)
    x = conv(x)
    x = conv(x, act=False)
    for _ in range(nres):
        f = triple(x, o96_ref)
        f = conv(f)
        f = conv(f)
        f = conv(f, act=False)
        x = x + f
    f = triple(x, o96_ref)
    f = conv(f)
    f = conv(f)
    out_ref[...] = conv(f, act=False)


def _mega(cat, o192, o96, flat_ws, nres):
    body = functools.partial(_mega_body, nres)
    return pl.pallas_call(
        body,
        out_shape=jax.ShapeDtypeStruct((96, 3), jnp.float32),
        interpret=INTERPRET,
    )(cat, o192, o96, *flat_ws)


def _triple_ws(p):
    out = []
    for l in ("1", "2", "3"):
        out.append(p["W" + l])
        out.append(p["b" + l].reshape(1, -1))
    return out


def kernel(sV, sFacesOneRingIdx, sW, sJ, rV, rFacesOneRingIdx, rW, rJ, skeleton_idx, params):
    b, n, k = sFacesOneRingIdx.shape
    j = sW.shape[1]
    p = 2 * b
    r = p * n

    v = jnp.concatenate([sV, rV], axis=0).reshape(r, 3)
    v = jnp.pad(v, ((0, 0), (0, 13)))
    idx = jnp.concatenate([sFacesOneRingIdx, rFacesOneRingIdx], axis=0)
    idx = idx.astype(jnp.int32) + (jnp.arange(p, dtype=jnp.int32) * n)[:, None, None]
    idx_kmaj = jnp.transpose(idx.reshape(r, k), (1, 0)).reshape(-1)

    geo = params["geo"]
    x = v
    feats = []
    for l in ("1", "2", "3"):
        w, bias = geo["W" + l], geo["b" + l]
        c = w.shape[0] // 2
        d = w.shape[1]
        wn, wc = w[:c], w[c:]
        cp = max(c, 16)
        wn = jnp.pad(wn, ((0, cp - c), (0, 0)))
        wc = jnp.pad(wc, ((0, cp - c), (0, 0)))
        g = _sc_gather(x, idx_kmaj)
        x = _edge_conv(x, g, wn, wc, bias, k)
        feats.append(x)
    f = jnp.concatenate(feats, axis=-1)

    wm = jnp.concatenate([sW, rW], axis=0)
    a = _a_matmul(wm, f.reshape(p, n, -1))
    jf = jnp.concatenate([sJ, rJ], axis=0)
    cat = jnp.concatenate([a, jf], axis=-1).reshape(p * j, -1)

    oh = jax.nn.one_hot(skeleton_idx, j, dtype=jnp.float32)
    oh_p = jnp.concatenate([oh, oh], axis=0)
    eye_p = jnp.eye(p, dtype=jnp.float32)
    eye_b = jnp.eye(b, dtype=jnp.float32)
    o192 = jnp.stack([
        jnp.einsum("pq,pjm->pjqm", eye_p, oh_p[:, :, kk, :]).reshape(p * j, p * j)
        for kk in range(4)
    ])
    o96 = jnp.stack([
        jnp.einsum("pq,pjm->pjqm", eye_b, oh[:, :, kk, :]).reshape(b * j, b * j)
        for kk in range(4)
    ])

    ws = []
    ws.extend(_triple_ws(params["skc"]))
    for nm in ("1", "2", "3"):
        ws.append(params["joint"]["W" + nm])
        ws.append(params["joint"]["b" + nm].reshape(1, -1))
    for blk in params["res"]:
        ws.extend(_triple_ws(blk["sk"]))
        for nm in ("1", "2", "3"):
            ws.append(blk["W" + nm])
            ws.append(blk["b" + nm].reshape(1, -1))
    ws.extend(_triple_ws(params["last"]["sk"]))
    for nm in ("1", "2", "3"):
        ws.append(params["last"]["W" + nm])
        ws.append(params["last"]["b" + nm].reshape(1, -1))

    out = _mega(cat, o192, o96, ws, len(params["res"]))
    return out.reshape(b, j, 3)


# speedup vs baseline: 1.5637x; 1.5637x over previous
import functools

import jax
import jax.numpy as jnp
from jax import lax
from jax.experimental import pallas as pl
from jax.experimental.pallas import tpu as pltpu
from jax.experimental.pallas import tpu_sc as plsc


def _lrelu(v):
    return jnp.where(v > 0, v, v * 0.2)


def _dot(a, b, prec=None):
    return jnp.dot(a, b, preferred_element_type=jnp.float32, precision=prec)


_SC_CH = 128


def _sc_gather(table, idx_flat):
    e = idx_flat.shape[0]
    c = table.shape[1]
    nw = 32
    per_w = e // nw
    n_chunks = per_w // _SC_CH
    n_pairs = n_chunks // 2
    mesh = plsc.VectorSubcoreMesh(core_axis_name="c", subcore_axis_name="s")

    @functools.partial(
        pl.kernel,
        mesh=mesh,
        out_type=jax.ShapeDtypeStruct((e, c), jnp.float32),
        scratch_types=[
            pltpu.VMEM((per_w,), jnp.int32),
            pltpu.VMEM((_SC_CH, c), jnp.float32),
            pltpu.VMEM((_SC_CH, c), jnp.float32),
            pltpu.VMEM((_SC_CH, c), jnp.float32),
            pltpu.VMEM((_SC_CH, c), jnp.float32),
            pltpu.SemaphoreType.DMA,
            pltpu.SemaphoreType.DMA,
            pltpu.SemaphoreType.DMA,
            pltpu.SemaphoreType.DMA,
            pltpu.SemaphoreType.DMA,
            pltpu.SemaphoreType.DMA,
            pltpu.SemaphoreType.DMA,
            pltpu.SemaphoreType.DMA,
        ],
    )
    def gk(table_hbm, idx_hbm, out_hbm, idx_all, r0, r1, r2, r3,
           g0, g1, g2, g3, s0, s1, s2, s3):
        wid = lax.axis_index("s") * 2 + lax.axis_index("c")
        base0 = wid * per_w
        pltpu.sync_copy(idx_hbm.at[pl.ds(base0, per_w)], idx_all)
        rows = (r0, r1, r2, r3)
        gsem = (g0, g1, g2, g3)
        ssem = (s0, s1, s2, s3)

        def g_start(ci, j):
            pltpu.async_copy(table_hbm.at[idx_all.at[pl.ds(ci * _SC_CH, _SC_CH)]], rows[j], gsem[j])

        def g_wait(ci, j):
            pltpu.make_async_copy(
                table_hbm.at[idx_all.at[pl.ds(ci * _SC_CH, _SC_CH)]], rows[j], gsem[j]
            ).wait()

        def s_start(ci, j):
            pltpu.async_copy(rows[j], out_hbm.at[pl.ds(base0 + ci * _SC_CH, _SC_CH)], ssem[j])

        def s_wait(ci, j):
            pltpu.make_async_copy(
                rows[j], out_hbm.at[pl.ds(base0 + ci * _SC_CH, _SC_CH)], ssem[j]
            ).wait()

        for j in range(4):
            g_start(j, j)

        n_quads = n_chunks // 4

        def body(q, carry):
            c = 4 * q - 4
            for j in range(4):
                g_wait(c + j, j)
                s_start(c + j, j)
            for j in range(4):
                s_wait(c + j, j)
                g_start(4 * q + j, j)
            return carry

        lax.fori_loop(1, n_quads, body, 0)
        c = 4 * n_quads - 4
        for j in range(4):
            g_wait(c + j, j)
            s_start(c + j, j)
        for j in range(4):
            s_wait(c + j, j)

    return gk(table, idx_flat)


def _edge_body(k, x_ref, g_ref, wn_ref, wc_ref, b_ref, o_ref):
    x = x_ref[...]
    zc = _dot(x, wc_ref[...]) + b_ref[...]
    m = None
    for kk in range(k):
        h = _dot(g_ref[kk] - x, wn_ref[...]) + zc
        m = h if m is None else jnp.maximum(m, h)
    o_ref[...] = _lrelu(m)


def _edge_conv(x, g, wn, wc, b, k, br=512):
    r, c = x.shape
    d = wn.shape[1]
    grid = r // br
    body = functools.partial(_edge_body, k)
    return pl.pallas_call(
        body,
        grid=(grid,),
        in_specs=[
            pl.BlockSpec((br, c), lambda i: (i, 0)),
            pl.BlockSpec((k, br, c), lambda i: (0, i, 0)),
            pl.BlockSpec((c, d), lambda i: (0, 0)),
            pl.BlockSpec((c, d), lambda i: (0, 0)),
            pl.BlockSpec((1, d), lambda i: (0, 0)),
        ],
        out_specs=pl.BlockSpec((br, d), lambda i: (i, 0)),
        out_shape=jax.ShapeDtypeStruct((r, d), jnp.float32),
    )(x, g.reshape(k, r, c), wn, wc, b.reshape(1, d))


def _a_body(d1, wm_ref, f1_ref, f2_ref, f3_ref, a_ref):
    wm = wm_ref[0]
    s = jnp.sum(wm, axis=-1, keepdims=True) + 1e-05
    a_ref[0, :, 0:d1] = _dot(wm, f1_ref[...][:, :d1]) / s
    d2 = f2_ref.shape[-1]
    a_ref[0, :, d1:d1 + d2] = _dot(wm, f2_ref[...]) / s
    d3 = f3_ref.shape[-1]
    a_ref[0, :, d1 + d2:d1 + d2 + d3] = _dot(wm, f3_ref[...]) / s


def _a_matmul(wm, f1, d1, f2, f3):
    p, j, n = wm.shape
    dtot = d1 + f2.shape[-1] + f3.shape[-1]
    c1 = f1.shape[-1]
    body = functools.partial(_a_body, d1)
    return pl.pallas_call(
        body,
        grid=(p,),
        in_specs=[
            pl.BlockSpec((1, j, n), lambda i: (i, 0, 0)),
            pl.BlockSpec((n, c1), lambda i: (i, 0)),
            pl.BlockSpec((n, f2.shape[-1]), lambda i: (i, 0)),
            pl.BlockSpec((n, f3.shape[-1]), lambda i: (i, 0)),
        ],
        out_specs=pl.BlockSpec((1, j, dtot), lambda i: (i, 0, 0)),
        out_shape=jax.ShapeDtypeStruct((p, j, dtot), jnp.float32),
    )(wm, f1, f2, f3)


def _mega_body(nres, cat_ref, o192_ref, o96_ref, *refs):
    refs = list(refs)
    out_ref = refs.pop()
    cur = [0]

    def take():
        v = refs[cur[0]]
        cur[0] += 1
        return v

    def conv(x, act=True):
        w, b = take()[...], take()[...]
        v = _dot(x, w) + b
        return _lrelu(v) if act else v

    def triple(x, o_ref):
        outs = []
        for _ in range(3):
            c = x.shape[-1]
            w, b = take()[...], take()[...]
            wn, wc = w[:c], w[c:]
            zc = _dot(x, wc) + b
            m = None
            for kk in range(4):
                nb = _dot(o_ref[kk], x, prec=lax.Precision.HIGHEST)
                h = _dot(nb - x, wn) + zc
                m = h if m is None else jnp.maximum(m, h)
            x = _lrelu(m)
            outs.append(x)
        return jnp.concatenate(outs, axis=-1)

    f192 = triple(cat_ref[...], o192_ref)
    x = jnp.concatenate([f192[:96], f192[96:]], axis=-1)
    x = conv(x)
    x = conv(x)
    x = conv(x, act=False)
    for _ in range(nres):
        f = triple(x, o96_ref)
        f = conv(f)
        f = conv(f)
        f = conv(f, act=False)
        x = x + f
    f = triple(x, o96_ref)
    f = conv(f)
    f = conv(f)
    out_ref[...] = conv(f, act=False)


def _mega(cat, o192, o96, flat_ws, nres):
    body = functools.partial(_mega_body, nres)
    return pl.pallas_call(
        body,
        out_shape=jax.ShapeDtypeStruct((96, 3), jnp.float32),
    )(cat, o192, o96, *flat_ws)


def _triple_ws(p):
    out = []
    for l in ("1", "2", "3"):
        out.append(p["W" + l])
        out.append(p["b" + l].reshape(1, -1))
    return out


def kernel(sV, sFacesOneRingIdx, sW, sJ, rV, rFacesOneRingIdx, rW, rJ, skeleton_idx, params):
    b, n, k = sFacesOneRingIdx.shape
    j = sW.shape[1]
    p = 2 * b
    r = p * n

    geo = params["geo"]
    rb = b * n

    def geo_branch(vb, idxb, wmb):
        x = jnp.pad(vb.reshape(rb, 3), ((0, 0), (0, 125)))
        idxf = idxb.astype(jnp.int32) + (jnp.arange(b, dtype=jnp.int32) * n)[:, None, None]
        idx_kmaj = jnp.transpose(idxf.reshape(rb, k), (1, 0)).reshape(-1)
        feats = []
        d1 = 0
        for l in ("1", "2", "3"):
            w, bias = geo["W" + l], geo["b" + l]
            c = w.shape[0] // 2
            d = w.shape[1]
            wn, wc = w[:c], w[c:]
            cp = max(c, 128)
            dp = max(d, 128)
            wn = jnp.pad(wn, ((0, cp - c), (0, dp - d)))
            wc = jnp.pad(wc, ((0, cp - c), (0, dp - d)))
            bias = jnp.pad(bias, (0, dp - d))
            g = _sc_gather(x, idx_kmaj)
            x = _edge_conv(x, g, wn, wc, bias, k)
            feats.append(x)
            if l == "1":
                d1 = d
        return _a_matmul(wmb, feats[0], d1, feats[1], feats[2])

    a_s = geo_branch(sV, sFacesOneRingIdx, sW)
    a_r = geo_branch(rV, rFacesOneRingIdx, rW)
    a = jnp.concatenate([a_s, a_r], axis=0)
    jf = jnp.concatenate([sJ, rJ], axis=0)
    cat = jnp.concatenate([a, jf], axis=-1).reshape(p * j, -1)

    oh = jax.nn.one_hot(skeleton_idx, j, dtype=jnp.float32)
    oh_p = jnp.concatenate([oh, oh], axis=0)
    eye_p = jnp.eye(p, dtype=jnp.float32)
    eye_b = jnp.eye(b, dtype=jnp.float32)
    o192 = jnp.stack([
        jnp.einsum("pq,pjm->pjqm", eye_p, oh_p[:, :, kk, :]).reshape(p * j, p * j)
        for kk in range(4)
    ])
    o96 = jnp.stack([
        jnp.einsum("pq,pjm->pjqm", eye_b, oh[:, :, kk, :]).reshape(b * j, b * j)
        for kk in range(4)
    ])

    ws = []
    ws.extend(_triple_ws(params["skc"]))
    for nm in ("1", "2", "3"):
        ws.append(params["joint"]["W" + nm])
        ws.append(params["joint"]["b" + nm].reshape(1, -1))
    for blk in params["res"]:
        ws.extend(_triple_ws(blk["sk"]))
        for nm in ("1", "2", "3"):
            ws.append(blk["W" + nm])
            ws.append(blk["b" + nm].reshape(1, -1))
    ws.extend(_triple_ws(params["last"]["sk"]))
    for nm in ("1", "2", "3"):
        ws.append(params["last"]["W" + nm])
        ws.append(params["last"]["b" + nm].reshape(1, -1))

    out = _mega(cat, o192, o96, ws, len(params["res"]))
    return out.reshape(b, j, 3)


# speedup vs baseline: 1.5984x; 1.0222x over previous
import functools

import jax
import jax.numpy as jnp
from jax import lax
from jax.experimental import pallas as pl
from jax.experimental.pallas import tpu as pltpu
from jax.experimental.pallas import tpu_sc as plsc


def _lrelu(v):
    return jnp.where(v > 0, v, v * 0.2)


def _dot(a, b, prec=None):
    return jnp.dot(a, b, preferred_element_type=jnp.float32, precision=prec)


_SC_CH = 128


def _sc_gather(table, idx_flat):
    e = idx_flat.shape[0]
    c = table.shape[1]
    nw = 32
    per_w = e // nw
    n_chunks = per_w // _SC_CH
    n_pairs = n_chunks // 2
    mesh = plsc.VectorSubcoreMesh(core_axis_name="c", subcore_axis_name="s")

    @functools.partial(
        pl.kernel,
        mesh=mesh,
        out_type=jax.ShapeDtypeStruct((e, c), jnp.float32),
        scratch_types=[
            pltpu.VMEM((per_w,), jnp.int32),
            pltpu.VMEM((_SC_CH, c), jnp.float32),
            pltpu.VMEM((_SC_CH, c), jnp.float32),
            pltpu.SemaphoreType.DMA,
            pltpu.SemaphoreType.DMA,
            pltpu.SemaphoreType.DMA,
            pltpu.SemaphoreType.DMA,
        ],
    )
    def gk(table_hbm, idx_hbm, out_hbm, idx_all, rows0, rows1, gsem0, gsem1, ssem0, ssem1):
        wid = lax.axis_index("s") * 2 + lax.axis_index("c")
        base0 = wid * per_w
        pltpu.sync_copy(idx_hbm.at[pl.ds(base0, per_w)], idx_all)

        def g_start(ci, rows, gsem):
            pltpu.async_copy(table_hbm.at[idx_all.at[pl.ds(ci * _SC_CH, _SC_CH)]], rows, gsem)

        def g_wait(ci, rows, gsem):
            pltpu.make_async_copy(
                table_hbm.at[idx_all.at[pl.ds(ci * _SC_CH, _SC_CH)]], rows, gsem
            ).wait()

        def s_start(ci, rows, ssem):
            pltpu.async_copy(rows, out_hbm.at[pl.ds(base0 + ci * _SC_CH, _SC_CH)], ssem)

        def s_wait(ci, rows, ssem):
            pltpu.make_async_copy(
                rows, out_hbm.at[pl.ds(base0 + ci * _SC_CH, _SC_CH)], ssem
            ).wait()

        g_start(0, rows0, gsem0)
        g_start(1, rows1, gsem1)

        def body(g2, carry):
            c0 = 2 * g2 - 2
            g_wait(c0, rows0, gsem0)
            s_start(c0, rows0, ssem0)
            s_wait(c0, rows0, ssem0)
            g_start(2 * g2, rows0, gsem0)
            g_wait(c0 + 1, rows1, gsem1)
            s_start(c0 + 1, rows1, ssem1)
            s_wait(c0 + 1, rows1, ssem1)
            g_start(2 * g2 + 1, rows1, gsem1)
            return carry

        lax.fori_loop(1, n_pairs, body, 0)
        c0 = 2 * n_pairs - 2
        g_wait(c0, rows0, gsem0)
        s_start(c0, rows0, ssem0)
        g_wait(c0 + 1, rows1, gsem1)
        s_start(c0 + 1, rows1, ssem1)
        s_wait(c0, rows0, ssem0)
        s_wait(c0 + 1, rows1, ssem1)

    return gk(table, idx_flat)


def _edge_body(k, x_ref, g_ref, wn_ref, wc_ref, b_ref, o_ref):
    x = x_ref[...]
    zc = _dot(x, wc_ref[...]) + b_ref[...]
    m = None
    for kk in range(k):
        h = _dot(g_ref[kk] - x, wn_ref[...]) + zc
        m = h if m is None else jnp.maximum(m, h)
    o_ref[...] = _lrelu(m)


def _edge_conv(x, g, wn, wc, b, k, br=512):
    r, c = x.shape
    d = wn.shape[1]
    grid = r // br
    body = functools.partial(_edge_body, k)
    return pl.pallas_call(
        body,
        grid=(grid,),
        in_specs=[
            pl.BlockSpec((br, c), lambda i: (i, 0)),
            pl.BlockSpec((k, br, c), lambda i: (0, i, 0)),
            pl.BlockSpec((c, d), lambda i: (0, 0)),
            pl.BlockSpec((c, d), lambda i: (0, 0)),
            pl.BlockSpec((1, d), lambda i: (0, 0)),
        ],
        out_specs=pl.BlockSpec((br, d), lambda i: (i, 0)),
        out_shape=jax.ShapeDtypeStruct((r, d), jnp.float32),
    )(x, g.reshape(k, r, c), wn, wc, b.reshape(1, d))


def _a_body(d1, wm_ref, f1_ref, f2_ref, f3_ref, a_ref):
    wm = wm_ref[0]
    s = jnp.sum(wm, axis=-1, keepdims=True) + 1e-05
    a_ref[0, :, 0:d1] = _dot(wm, f1_ref[...][:, :d1]) / s
    d2 = f2_ref.shape[-1]
    a_ref[0, :, d1:d1 + d2] = _dot(wm, f2_ref[...]) / s
    d3 = f3_ref.shape[-1]
    a_ref[0, :, d1 + d2:d1 + d2 + d3] = _dot(wm, f3_ref[...]) / s


def _a_matmul(wm, f1, d1, f2, f3):
    p, j, n = wm.shape
    dtot = d1 + f2.shape[-1] + f3.shape[-1]
    c1 = f1.shape[-1]
    body = functools.partial(_a_body, d1)
    return pl.pallas_call(
        body,
        grid=(p,),
        in_specs=[
            pl.BlockSpec((1, j, n), lambda i: (i, 0, 0)),
            pl.BlockSpec((n, c1), lambda i: (i, 0)),
            pl.BlockSpec((n, f2.shape[-1]), lambda i: (i, 0)),
            pl.BlockSpec((n, f3.shape[-1]), lambda i: (i, 0)),
        ],
        out_specs=pl.BlockSpec((1, j, dtot), lambda i: (i, 0, 0)),
        out_shape=jax.ShapeDtypeStruct((p, j, dtot), jnp.float32),
    )(wm, f1, f2, f3)


def _mega_body(nres, cat_ref, o192_ref, o96_ref, *refs):
    refs = list(refs)
    out_ref = refs.pop()
    cur = [0]

    def take():
        v = refs[cur[0]]
        cur[0] += 1
        return v

    def conv(x, act=True):
        w, b = take()[...], take()[...]
        v = _dot(x, w) + b
        return _lrelu(v) if act else v

    def triple(x, o_ref):
        outs = []
        for _ in range(3):
            c = x.shape[-1]
            w, b = take()[...], take()[...]
            wn, wc = w[:c], w[c:]
            zc = _dot(x, wc) + b
            m = None
            for kk in range(4):
                nb = _dot(o_ref[kk], x, prec=lax.Precision.HIGHEST)
                h = _dot(nb - x, wn) + zc
                m = h if m is None else jnp.maximum(m, h)
            x = _lrelu(m)
            outs.append(x)
        return jnp.concatenate(outs, axis=-1)

    f192 = triple(cat_ref[...], o192_ref)
    x = jnp.concatenate([f192[:96], f192[96:]], axis=-1)
    x = conv(x)
    x = conv(x)
    x = conv(x, act=False)
    for _ in range(nres):
        f = triple(x, o96_ref)
        f = conv(f)
        f = conv(f)
        f = conv(f, act=False)
        x = x + f
    f = triple(x, o96_ref)
    f = conv(f)
    f = conv(f)
    out_ref[...] = conv(f, act=False)


def _mega(cat, o192, o96, flat_ws, nres):
    body = functools.partial(_mega_body, nres)
    return pl.pallas_call(
        body,
        out_shape=jax.ShapeDtypeStruct((96, 3), jnp.float32),
    )(cat, o192, o96, *flat_ws)


def _triple_ws(p):
    out = []
    for l in ("1", "2", "3"):
        out.append(p["W" + l])
        out.append(p["b" + l].reshape(1, -1))
    return out


def kernel(sV, sFacesOneRingIdx, sW, sJ, rV, rFacesOneRingIdx, rW, rJ, skeleton_idx, params):
    b, n, k = sFacesOneRingIdx.shape
    j = sW.shape[1]
    p = 2 * b
    r = p * n

    geo = params["geo"]
    rb = b * n

    def geo_branch(vb, idxb, wmb):
        x = jnp.pad(vb.reshape(rb, 3), ((0, 0), (0, 125)))
        idxf = idxb.astype(jnp.int32) + (jnp.arange(b, dtype=jnp.int32) * n)[:, None, None]
        idx_kmaj = jnp.transpose(idxf.reshape(rb, k), (1, 0)).reshape(-1)
        feats = []
        d1 = 0
        for l in ("1", "2", "3"):
            w, bias = geo["W" + l], geo["b" + l]
            c = w.shape[0] // 2
            d = w.shape[1]
            wn, wc = w[:c], w[c:]
            cp = max(c, 128)
            dp = max(d, 128)
            wn = jnp.pad(wn, ((0, cp - c), (0, dp - d)))
            wc = jnp.pad(wc, ((0, cp - c), (0, dp - d)))
            bias = jnp.pad(bias, (0, dp - d))
            g = _sc_gather(x, idx_kmaj)
            x = _edge_conv(x, g, wn, wc, bias, k)
            feats.append(x)
            if l == "1":
                d1 = d
        return _a_matmul(wmb, feats[0], d1, feats[1], feats[2])

    a_s = geo_branch(sV, sFacesOneRingIdx, sW)
    a_r = geo_branch(rV, rFacesOneRingIdx, rW)
    a = jnp.concatenate([a_s, a_r], axis=0)
    jf = jnp.concatenate([sJ, rJ], axis=0)
    cat = jnp.concatenate([a, jf], axis=-1).reshape(p * j, -1)

    oh = jax.nn.one_hot(skeleton_idx, j, dtype=jnp.float32)
    oh_p = jnp.concatenate([oh, oh], axis=0)
    eye_p = jnp.eye(p, dtype=jnp.float32)
    eye_b = jnp.eye(b, dtype=jnp.float32)
    o192 = jnp.stack([
        jnp.einsum("pq,pjm->pjqm", eye_p, oh_p[:, :, kk, :]).reshape(p * j, p * j)
        for kk in range(4)
    ])
    o96 = jnp.stack([
        jnp.einsum("pq,pjm->pjqm", eye_b, oh[:, :, kk, :]).reshape(b * j, b * j)
        for kk in range(4)
    ])

    ws = []
    ws.extend(_triple_ws(params["skc"]))
    for nm in ("1", "2", "3"):
        ws.append(params["joint"]["W" + nm])
        ws.append(params["joint"]["b" + nm].reshape(1, -1))
    for blk in params["res"]:
        ws.extend(_triple_ws(blk["sk"]))
        for nm in ("1", "2", "3"):
            ws.append(blk["W" + nm])
            ws.append(blk["b" + nm].reshape(1, -1))
    ws.extend(_triple_ws(params["last"]["sk"]))
    for nm in ("1", "2", "3"):
        ws.append(params["last"]["W" + nm])
        ws.append(params["last"]["b" + nm].reshape(1, -1))

    out = _mega(cat, o192, o96, ws, len(params["res"]))
    return out.reshape(b, j, 3)
